# baseline (speedup 1.0000x reference)
import functools

import jax
import jax.numpy as jnp
from jax import lax
from jax.experimental import pallas as pl
from jax.experimental.pallas import tpu as pltpu
from jax.experimental.pallas import tpu_sc as plsc

D = 256
K = 8192
T = 4096
L = 16
NC = 2
NS = 16
DC = D // L
RW = T // NS
SW = K // NS
CH = 128

_mesh = plsc.VectorSubcoreMesh(
    core_axis_name="c", subcore_axis_name="s", num_cores=NC, num_subcores=NS
)


def _accum_rows(buf, accs):
    def row_step(r, a):
        return tuple(a[cc] + buf[r, pl.ds(cc * L, L)] for cc in range(DC))
    return lax.fori_loop(0, CH, row_step, accs)


def _tree_hsum(accs, lane):
    idx_e = (lane % 8) * 2
    idx_o = idx_e + 1
    lo = lane < 8

    def combine(a, b):
        a_e = a.at[idx_e].get(mode="promise_in_bounds")
        a_o = a.at[idx_o].get(mode="promise_in_bounds")
        b_e = b.at[idx_e].get(mode="promise_in_bounds")
        b_o = b.at[idx_o].get(mode="promise_in_bounds")
        return jnp.where(lo, a_e + a_o, b_e + b_o)

    level = accs
    while len(level) > 1:
        level = [combine(level[2 * k], level[2 * k + 1])
                 for k in range(len(level) // 2)]
    return level[0]


@functools.partial(
    pl.kernel,
    out_type=(
        jax.ShapeDtypeStruct((L,), jnp.int32),
        jax.ShapeDtypeStruct((K,), jnp.float32),
        jax.ShapeDtypeStruct((NC * NS, D), jnp.float32),
        jax.ShapeDtypeStruct((NC * NS, L), jnp.float32),
    ),
    mesh=_mesh,
    scratch_types=[
        pltpu.VMEM((CH, D), jnp.float32),
        pltpu.VMEM((CH, D), jnp.float32),
        pltpu.VMEM((D,), jnp.float32),
        pltpu.VMEM((NS, D), jnp.float32),
        pltpu.VMEM((L,), jnp.float32),
        pltpu.VMEM((NS, L), jnp.float32),
        pltpu.VMEM((K // (NC * NS),), jnp.float32),
        pltpu.VMEM((L,), jnp.int32),
        pltpu.SemaphoreType.DMA,
        pltpu.SemaphoreType.DMA,
    ],
)
def _engram(hid_hbm, seed_hbm, usage_hbm, idx_hbm, usage_out_hbm,
            stage_q, stage_c,
            buf0, buf1, qrow, qtmp, crow, ctmp, usv, idxv,
            sem0, sem1):
    c = lax.axis_index("c")
    s = lax.axis_index("s")
    lane = lax.iota(jnp.int32, L)
    zeros = jnp.zeros((L,), jnp.float32)

    r0 = s * RW
    cp0 = pltpu.make_async_copy(hid_hbm.at[pl.ds(r0, CH)], buf0, sem0)
    cp0.start()
    cp1 = pltpu.make_async_copy(hid_hbm.at[pl.ds(r0 + CH, CH)], buf1, sem1)
    cp1.start()
    cp0.wait()
    accs = _accum_rows(buf0, (zeros,) * DC)
    cp1.wait()
    accs = _accum_rows(buf1, accs)
    for cc in range(DC):
        qrow[pl.ds(cc * L, L)] = accs[cc]
    pltpu.sync_copy(qrow, stage_q.at[c * NS + s])
    plsc.subcore_barrier()
    pltpu.sync_copy(stage_q.at[pl.ds(c * NS, NS)], qtmp)
    q = []
    inv_t = 1.0 / T
    for cc in range(DC):
        acc = qtmp[0, pl.ds(cc * L, L)]
        for r in range(1, NS):
            acc = acc + qtmp[r, pl.ds(cc * L, L)]
        q.append(acc * inv_t)

    sbase = s * SW
    cp0 = pltpu.make_async_copy(seed_hbm.at[pl.ds(sbase, CH)], buf0, sem0)
    cp0.start()
    cp1 = pltpu.make_async_copy(seed_hbm.at[pl.ds(sbase + CH, CH)], buf1, sem1)
    cp1.start()

    best_d = jnp.full((L,), jnp.inf, jnp.float32)
    best_i = jnp.zeros((L,), jnp.int32)

    def process_chunk(buf, base, bd, bi):
        def blk_step(b, carry):
            bd, bi = carry
            accs = []
            for j in range(L):
                row = b * L + j
                acc = None
                for cc in range(DC):
                    dv = buf[row, pl.ds(cc * L, L)] - q[cc]
                    acc = dv * dv if acc is None else acc + dv * dv
                accs.append(acc)
            dist = _tree_hsum(accs, lane)
            idx = base + b * L + lane
            better = dist < bd
            return jnp.where(better, dist, bd), jnp.where(better, idx, bi)
        return lax.fori_loop(0, CH // L, blk_step, (bd, bi))

    for ch in range(SW // CH):
        buf = buf0 if ch % 2 == 0 else buf1
        sem = sem0 if ch % 2 == 0 else sem1
        cp = pltpu.make_async_copy(
            seed_hbm.at[pl.ds(sbase + ch * CH, CH)], buf, sem)
        cp.wait()
        best_d, best_i = process_chunk(buf, sbase + ch * CH, best_d, best_i)
        nxt = ch + 2
        if nxt < SW // CH:
            pltpu.make_async_copy(
                seed_hbm.at[pl.ds(sbase + nxt * CH, CH)], buf, sem).start()

    d_best = best_d[0]
    i_best = best_i[0]
    for l in range(1, L):
        dl = best_d[l]
        il = best_i[l]
        better = dl < d_best
        d_best = lax.select(better, dl, d_best)
        i_best = lax.select(better, il, i_best)
    crow[...] = jnp.where(lane == 0, d_best,
                          jnp.where(lane == 1, i_best.astype(jnp.float32), 0.0))
    pltpu.sync_copy(crow, stage_c.at[c * NS + s])
    plsc.subcore_barrier()
    pltpu.sync_copy(stage_c.at[pl.ds(c * NS, NS)], ctmp)

    def merge_step(i, carry):
        gd, gi = carry
        v = ctmp[i, pl.ds(0, L)]
        d = v[0]
        ind = v[1]
        better = d < gd
        return (lax.select(better, d, gd), lax.select(better, ind, gi))

    _, gi_f = lax.fori_loop(0, NS, merge_step,
                            (jnp.float32(jnp.inf), jnp.float32(0.0)))
    winner = gi_f.astype(jnp.int32)

    wid = s * NC + c
    us_n = K // (NC * NS)
    ubase = wid * us_n
    pltpu.sync_copy(usage_hbm.at[pl.ds(ubase, us_n)], usv)
    off = winner - ubase

    @pl.when((off >= 0) & (off < us_n))
    def _():
        blk = (off // L) * L
        vec = usv[pl.ds(pl.multiple_of(blk, L), L)]
        usv[pl.ds(pl.multiple_of(blk, L), L)] = vec + jnp.where(
            lane == off - blk, 1.0, 0.0)

    pltpu.sync_copy(usv, usage_out_hbm.at[pl.ds(ubase, us_n)])

    @pl.when((s == 0) & (c == 0))
    def _():
        idxv[...] = jnp.full((L,), winner, jnp.int32)
        pltpu.sync_copy(idxv, idx_hbm)


@jax.jit
def kernel(hidden_state, seed_bank, usage_frequency):
    idx16, usage_new, _, _ = _engram(hidden_state, seed_bank, usage_frequency)
    return idx16[:1], usage_new
